# baseline (speedup 1.0000x reference)
import functools

import jax
import jax.numpy as jnp
from jax import lax
from jax.experimental import pallas as pl
from jax.experimental.pallas import tpu as pltpu
from jax.experimental.pallas import tpu_sc as plsc

B, L, D = 1024, 20, 128
M = B * L
K_HARD = 16
T_TOTAL = 512
NUM_HARD = 153
NUM_INBATCH = T_TOTAL - NUM_HARD
INV_TEMP = 20.0
IDX_PAD = 368
N_GROUPS = IDX_PAD // 16
TAIL_VALID = NUM_INBATCH - (N_GROUPS - 1) * 16

_SC_INFO = plsc.get_sparse_core_info()
NC = _SC_INFO.num_cores
NS = _SC_INFO.num_subcores
NW = NC * NS


def _norm_body(q_ref, k_ref, qn_ref, qb_ref, kb_ref, l0_ref):
    q = q_ref[...]
    k = k_ref[...]
    qn = q / jnp.maximum(jnp.sqrt(jnp.sum(q * q, axis=1, keepdims=True)), 1e-12)
    kn = k / jnp.maximum(jnp.sqrt(jnp.sum(k * k, axis=1, keepdims=True)), 1e-12)
    qn_ref[...] = qn
    qb_ref[...] = qn.astype(jnp.bfloat16)
    kb_ref[...] = kn.astype(jnp.bfloat16)
    l0_ref[...] = jnp.sum(qn * kn, axis=1) * INV_TEMP


def _normalize(q, k):
    bm = 2048
    return pl.pallas_call(
        _norm_body,
        grid=(M // bm,),
        in_specs=[
            pl.BlockSpec((bm, D), lambda i: (i, 0)),
            pl.BlockSpec((bm, D), lambda i: (i, 0)),
        ],
        out_specs=[
            pl.BlockSpec((bm, D), lambda i: (i, 0)),
            pl.BlockSpec((bm, D), lambda i: (i, 0)),
            pl.BlockSpec((bm, D), lambda i: (i, 0)),
            pl.BlockSpec((bm,), lambda i: (i,)),
        ],
        out_shape=[
            jax.ShapeDtypeStruct((M, D), jnp.float32),
            jax.ShapeDtypeStruct((M, D), jnp.bfloat16),
            jax.ShapeDtypeStruct((M, D), jnp.bfloat16),
            jax.ShapeDtypeStruct((M,), jnp.float32),
        ],
    )(q, k)


def _matmul_body(a_ref, b_ref, o_ref):
    o_ref[...] = lax.dot_general(
        a_ref[...], b_ref[...],
        (((1,), (1,)), ((), ())),
        preferred_element_type=jnp.float32,
    ) * INV_TEMP


def _similarity(qb, kb):
    bm, bn = 1024, 1024
    return pl.pallas_call(
        _matmul_body,
        grid=(M // bm, M // bn),
        in_specs=[
            pl.BlockSpec((bm, D), lambda i, j: (i, 0)),
            pl.BlockSpec((bn, D), lambda i, j: (j, 0)),
        ],
        out_specs=pl.BlockSpec((bm, bn), lambda i, j: (i, j)),
        out_shape=jax.ShapeDtypeStruct((M, M), jnp.float32),
    )(qb, kb)


_G_TOTAL = M * K_HARD
_G_PER_W = _G_TOTAL // NW
_G_CHUNK = 128
_G_STEPS = _G_PER_W // _G_CHUNK


def _hard_gather_body(table_hbm, ids_hbm, out_hbm, idx_v, rows_v, sem):
    wid = lax.axis_index("s") * NC + lax.axis_index("c")
    wbase = wid * _G_PER_W

    def step(c, _):
        base = wbase + c * _G_CHUNK
        pltpu.sync_copy(ids_hbm.at[pl.ds(base, _G_CHUNK)], idx_v)
        pltpu.async_copy(table_hbm.at[idx_v], rows_v, sem).wait()
        pltpu.sync_copy(rows_v, out_hbm.at[pl.ds(base, _G_CHUNK)])
        return _

    lax.fori_loop(0, _G_STEPS, step, None)


def _hard_gather(item_table, ids_flat):
    mesh = plsc.VectorSubcoreMesh(core_axis_name="c", subcore_axis_name="s")
    f = pl.kernel(
        _hard_gather_body,
        out_type=jax.ShapeDtypeStruct((_G_TOTAL, D), jnp.float32),
        mesh=mesh,
        scratch_types=[
            pltpu.VMEM((_G_CHUNK,), jnp.int32),
            pltpu.VMEM((_G_CHUNK, D), jnp.float32),
            pltpu.SemaphoreType.DMA,
        ],
    )
    return f(item_table, ids_flat)


_E_TOTAL = M * IDX_PAD
_E_PER_W = _E_TOTAL // NW
_E_CHUNK = 1024
_E_NFIRE = 23
_E_GRP = _E_CHUNK * _E_NFIRE
_E_NGRP = _E_PER_W // _E_GRP


def _inbatch_body(s_hbm, idx_hbm, out_hbm, idx_v, val_v, sem):
    wid = lax.axis_index("s") * NC + lax.axis_index("c")
    wbase = wid * _E_PER_W

    def group(g, _):
        base = wbase + g * _E_GRP
        pltpu.sync_copy(idx_hbm.at[pl.ds(base, _E_GRP)], idx_v)
        copies = [
            pltpu.async_copy(
                s_hbm.at[idx_v.at[pl.ds(c * _E_CHUNK, _E_CHUNK)]],
                val_v.at[pl.ds(c * _E_CHUNK, _E_CHUNK)],
                sem,
            )
            for c in range(_E_NFIRE)
        ]
        for cp in copies:
            cp.wait()
        pltpu.sync_copy(val_v, out_hbm.at[pl.ds(base, _E_GRP)])
        return _

    lax.fori_loop(0, _E_NGRP, group, None)


def _inbatch_gather(s_flat, flat_idx):
    mesh = plsc.VectorSubcoreMesh(core_axis_name="c", subcore_axis_name="s")
    f = pl.kernel(
        _inbatch_body,
        out_type=jax.ShapeDtypeStruct((_E_TOTAL,), jnp.float32),
        mesh=mesh,
        scratch_types=[
            pltpu.VMEM((_E_GRP,), jnp.int32),
            pltpu.VMEM((_E_GRP,), jnp.float32),
            pltpu.SemaphoreType.DMA,
        ],
        compiler_params=pltpu.CompilerParams(needs_layout_passes=False),
    )
    return f(s_flat, flat_idx)


def _final_body(qn_ref, h_ref, sidx_ref, l0_ref, inp_ref, mask_ref,
                num_ref, den_ref):
    @pl.when(pl.program_id(0) == 0)
    def _init():
        num_ref[...] = jnp.zeros((1, 1), jnp.float32)
        den_ref[...] = jnp.zeros((1, 1), jnp.float32)

    qn = qn_ref[...]
    h = h_ref[...]
    hn = h / jnp.maximum(
        jnp.sqrt(jnp.sum(h * h, axis=2, keepdims=True)), 1e-12)
    l16 = jnp.sum(qn[:, None, :] * hn, axis=2) * INV_TEMP
    e16 = jnp.exp(l16)
    sidx = sidx_ref[...]
    hard_sum = jnp.zeros(e16.shape[:1], jnp.float32)
    for j in range(K_HARD):
        cnt = jnp.sum((sidx == j).astype(jnp.float32), axis=1)
        hard_sum = hard_sum + cnt * e16[:, j]
    v = inp_ref[...]
    lane = lax.broadcasted_iota(jnp.int32, v.shape, 1)
    in_sum = jnp.sum(jnp.where(lane < NUM_INBATCH, jnp.exp(v), 0.0), axis=1)
    l0 = l0_ref[...]
    mask = (mask_ref[...] > 0.5).astype(jnp.float32)
    loss = jnp.log(jnp.exp(l0) + hard_sum + in_sum) - l0
    num_ref[...] += jnp.sum(loss * mask).reshape(1, 1)
    den_ref[...] += jnp.sum(mask).reshape(1, 1)


def _finalize(qn, hard_rows, sample_idx, l0, in_part, mask_flat):
    bf = 1024
    return pl.pallas_call(
        _final_body,
        grid=(M // bf,),
        in_specs=[
            pl.BlockSpec((bf, D), lambda i: (i, 0)),
            pl.BlockSpec((bf, K_HARD, D), lambda i: (i, 0, 0)),
            pl.BlockSpec((bf, NUM_HARD), lambda i: (i, 0)),
            pl.BlockSpec((bf,), lambda i: (i,)),
            pl.BlockSpec((bf, IDX_PAD), lambda i: (i, 0)),
            pl.BlockSpec((bf,), lambda i: (i,)),
        ],
        out_specs=[
            pl.BlockSpec((1, 1), lambda i: (0, 0)),
            pl.BlockSpec((1, 1), lambda i: (0, 0)),
        ],
        out_shape=[
            jax.ShapeDtypeStruct((1, 1), jnp.float32),
            jax.ShapeDtypeStruct((1, 1), jnp.float32),
        ],
    )(qn, hard_rows, sample_idx, l0, in_part, mask_flat)


def kernel(user_embs, pos_embs, neg_embs, hard_negatives, loss_mask, item_table):
    del neg_embs
    q = user_embs.reshape(M, D)
    k = pos_embs.reshape(M, D)
    ids_flat = hard_negatives.reshape(M * K_HARD)

    skey = jax.random.key(42)
    k1, k2 = jax.random.split(skey)
    sample_idx = jax.random.randint(k1, (M, NUM_HARD), 0, K_HARD)
    rows = jnp.arange(M)
    r = jax.random.randint(k2, (M, NUM_INBATCH), 0, M - 1)
    inbatch_idx = r + (r >= rows[:, None]).astype(r.dtype)
    flat_idx = inbatch_idx.astype(jnp.int32) + rows[:, None].astype(jnp.int32) * M
    flat_idx = jnp.pad(flat_idx, ((0, 0), (0, IDX_PAD - NUM_INBATCH)))

    qn, qb, kb, l0 = _normalize(q, k)
    hard_rows = _hard_gather(item_table, ids_flat)
    s = _similarity(qb, kb)
    in_vals = _inbatch_gather(s.reshape(M * M), flat_idx.reshape(_E_TOTAL))
    num, den = _finalize(
        qn,
        hard_rows.reshape(M, K_HARD, D),
        sample_idx.astype(jnp.int32),
        l0,
        in_vals.reshape(M, IDX_PAD),
        loss_mask.reshape(M),
    )
    return num[0, 0] / den[0, 0]


# speedup vs baseline: 1.4950x; 1.4950x over previous
import functools

import jax
import jax.numpy as jnp
from jax import lax
from jax.experimental import pallas as pl
from jax.experimental.pallas import tpu as pltpu
from jax.experimental.pallas import tpu_sc as plsc

B, L, D = 1024, 20, 128
M = B * L
K_HARD = 16
T_TOTAL = 512
NUM_HARD = 153
NUM_INBATCH = T_TOTAL - NUM_HARD
INV_TEMP = 20.0
IDX_PAD = 368
N_GROUPS = IDX_PAD // 16
TAIL_VALID = NUM_INBATCH - (N_GROUPS - 1) * 16

_SC_INFO = plsc.get_sparse_core_info()
NC = _SC_INFO.num_cores
NS = _SC_INFO.num_subcores
NW = NC * NS


def _norm_body(q_ref, k_ref, qn_ref, qb_ref, kb_ref, l0_ref):
    q = q_ref[...]
    k = k_ref[...]
    qn = q / jnp.maximum(jnp.sqrt(jnp.sum(q * q, axis=1, keepdims=True)), 1e-12)
    kn = k / jnp.maximum(jnp.sqrt(jnp.sum(k * k, axis=1, keepdims=True)), 1e-12)
    qn_ref[...] = qn
    qb_ref[...] = qn.astype(jnp.bfloat16)
    kb_ref[...] = kn.astype(jnp.bfloat16)
    l0_ref[...] = jnp.sum(qn * kn, axis=1) * INV_TEMP


def _normalize(q, k):
    bm = 2048
    return pl.pallas_call(
        _norm_body,
        grid=(M // bm,),
        in_specs=[
            pl.BlockSpec((bm, D), lambda i: (i, 0)),
            pl.BlockSpec((bm, D), lambda i: (i, 0)),
        ],
        out_specs=[
            pl.BlockSpec((bm, D), lambda i: (i, 0)),
            pl.BlockSpec((bm, D), lambda i: (i, 0)),
            pl.BlockSpec((bm, D), lambda i: (i, 0)),
            pl.BlockSpec((bm,), lambda i: (i,)),
        ],
        out_shape=[
            jax.ShapeDtypeStruct((M, D), jnp.float32),
            jax.ShapeDtypeStruct((M, D), jnp.bfloat16),
            jax.ShapeDtypeStruct((M, D), jnp.bfloat16),
            jax.ShapeDtypeStruct((M,), jnp.float32),
        ],
    )(q, k)


def _matmul_body(a_ref, b_ref, o_ref):
    o_ref[...] = lax.dot_general(
        a_ref[...], b_ref[...],
        (((1,), (1,)), ((), ())),
        preferred_element_type=jnp.float32,
    ) * INV_TEMP


def _similarity(qb, kb):
    bm, bn = 1024, 1024
    return pl.pallas_call(
        _matmul_body,
        grid=(M // bm, M // bn),
        in_specs=[
            pl.BlockSpec((bm, D), lambda i, j: (i, 0)),
            pl.BlockSpec((bn, D), lambda i, j: (j, 0)),
        ],
        out_specs=pl.BlockSpec((bm, bn), lambda i, j: (i, j)),
        out_shape=jax.ShapeDtypeStruct((M, M), jnp.float32),
    )(qb, kb)


_G_TOTAL = M * K_HARD
_G_PER_W = _G_TOTAL // NW
_G_CHUNK = 128
_G_STEPS = _G_PER_W // _G_CHUNK


def _hard_gather_body(table_hbm, ids_hbm, out_hbm, idx_v, rows_v, sem):
    wid = lax.axis_index("s") * NC + lax.axis_index("c")
    wbase = wid * _G_PER_W

    def step(c, _):
        base = wbase + c * _G_CHUNK
        pltpu.sync_copy(ids_hbm.at[pl.ds(base, _G_CHUNK)], idx_v)
        pltpu.async_copy(table_hbm.at[idx_v], rows_v, sem).wait()
        pltpu.sync_copy(rows_v, out_hbm.at[pl.ds(base, _G_CHUNK)])
        return _

    lax.fori_loop(0, _G_STEPS, step, None)


def _hard_gather(item_table, ids_flat):
    mesh = plsc.VectorSubcoreMesh(core_axis_name="c", subcore_axis_name="s")
    f = pl.kernel(
        _hard_gather_body,
        out_type=jax.ShapeDtypeStruct((_G_TOTAL, D), jnp.float32),
        mesh=mesh,
        scratch_types=[
            pltpu.VMEM((_G_CHUNK,), jnp.int32),
            pltpu.VMEM((_G_CHUNK, D), jnp.float32),
            pltpu.SemaphoreType.DMA,
        ],
    )
    return f(item_table, ids_flat)


_R_PER_W = M // NW
_R_GRP = 64
_R_NGRP = _R_PER_W // _R_GRP


def _inbatch_body(s_hbm, idx_hbm, out_hbm, srow_v, idx_v, acc_v,
                  sem0, sem1):
    wid = lax.axis_index("s") * NC + lax.axis_index("c")
    wbase = wid * _R_PER_W
    iota16 = lax.iota(jnp.int32, 16)
    tail_mask = iota16 < TAIL_VALID
    sems = (sem0, sem1)
    bufs = (srow_v.at[pl.ds(0, M)], srow_v.at[pl.ds(M, M)])

    pltpu.async_copy(s_hbm.at[wbase], bufs[0], sem0)

    def group(g, _):
        grp_base = g * _R_GRP
        pltpu.sync_copy(
            idx_hbm.at[pl.ds((wbase + grp_base) * IDX_PAD, _R_GRP * IDX_PAD)],
            idx_v)

        def pair(ph, _):
            for sub in range(2):
                gi = grp_base + ph * 2 + sub
                m = wbase + gi

                @pl.when(gi + 1 < _R_PER_W)
                def _prefetch():
                    pltpu.async_copy(s_hbm.at[m + 1], bufs[1 - sub],
                                     sems[1 - sub])

                pltpu.make_async_copy(s_hbm.at[0], bufs[sub],
                                      sems[sub]).wait()
                li = (ph * 2 + sub) * IDX_PAD
                acc = jnp.zeros((16,), jnp.float32)
                off = sub * M
                for g16 in range(N_GROUPS):
                    iv = idx_v[pl.ds(li + g16 * 16, 16)] + off
                    vals = plsc.load_gather(srow_v, [iv])
                    e = jnp.exp(vals)
                    if g16 == N_GROUPS - 1:
                        e = jnp.where(tail_mask, e, 0.0)
                    acc = acc + e
                acc_v[pl.ds((ph * 2 + sub) * 16, 16)] = acc
            return _

        lax.fori_loop(0, _R_GRP // 2, pair, None)
        pltpu.sync_copy(acc_v,
                        out_hbm.at[pl.ds((wbase + grp_base) * 16, _R_GRP * 16)])
        return _

    lax.fori_loop(0, _R_NGRP, group, None)


def _inbatch_sums(s, idx_pad_flat):
    mesh = plsc.VectorSubcoreMesh(core_axis_name="c", subcore_axis_name="s")
    f = pl.kernel(
        _inbatch_body,
        out_type=jax.ShapeDtypeStruct((M * 16,), jnp.float32),
        mesh=mesh,
        scratch_types=[
            pltpu.VMEM((2 * M,), jnp.float32),
            pltpu.VMEM((_R_GRP * IDX_PAD,), jnp.int32),
            pltpu.VMEM((_R_GRP * 16,), jnp.float32),
            pltpu.SemaphoreType.DMA,
            pltpu.SemaphoreType.DMA,
        ],
        compiler_params=pltpu.CompilerParams(needs_layout_passes=False),
    )
    return f(s, idx_pad_flat)


def _final_body(qn_ref, h_ref, sidx_ref, l0_ref, inp_ref, mask_ref,
                num_ref, den_ref):
    @pl.when(pl.program_id(0) == 0)
    def _init():
        num_ref[...] = jnp.zeros((1, 1), jnp.float32)
        den_ref[...] = jnp.zeros((1, 1), jnp.float32)

    qn = qn_ref[...]
    h = h_ref[...]
    hn = h / jnp.maximum(
        jnp.sqrt(jnp.sum(h * h, axis=2, keepdims=True)), 1e-12)
    l16 = jnp.sum(qn[:, None, :] * hn, axis=2) * INV_TEMP
    e16 = jnp.exp(l16)
    sidx = sidx_ref[...]
    hard_sum = jnp.zeros(e16.shape[:1], jnp.float32)
    for j in range(K_HARD):
        cnt = jnp.sum((sidx == j).astype(jnp.float32), axis=1)
        hard_sum = hard_sum + cnt * e16[:, j]
    in_sum = jnp.sum(inp_ref[...], axis=1)
    l0 = l0_ref[...]
    mask = (mask_ref[...] > 0.5).astype(jnp.float32)
    loss = jnp.log(jnp.exp(l0) + hard_sum + in_sum) - l0
    num_ref[...] += jnp.sum(loss * mask).reshape(1, 1)
    den_ref[...] += jnp.sum(mask).reshape(1, 1)


def _finalize(qn, hard_rows, sample_idx, l0, in_part, mask_flat):
    bf = 1024
    return pl.pallas_call(
        _final_body,
        grid=(M // bf,),
        in_specs=[
            pl.BlockSpec((bf, D), lambda i: (i, 0)),
            pl.BlockSpec((bf, K_HARD, D), lambda i: (i, 0, 0)),
            pl.BlockSpec((bf, NUM_HARD), lambda i: (i, 0)),
            pl.BlockSpec((bf,), lambda i: (i,)),
            pl.BlockSpec((bf, 16), lambda i: (i, 0)),
            pl.BlockSpec((bf,), lambda i: (i,)),
        ],
        out_specs=[
            pl.BlockSpec((1, 1), lambda i: (0, 0)),
            pl.BlockSpec((1, 1), lambda i: (0, 0)),
        ],
        out_shape=[
            jax.ShapeDtypeStruct((1, 1), jnp.float32),
            jax.ShapeDtypeStruct((1, 1), jnp.float32),
        ],
    )(qn, hard_rows, sample_idx, l0, in_part, mask_flat)


def kernel(user_embs, pos_embs, neg_embs, hard_negatives, loss_mask, item_table):
    del neg_embs
    q = user_embs.reshape(M, D)
    k = pos_embs.reshape(M, D)
    ids_flat = hard_negatives.reshape(M * K_HARD)

    skey = jax.random.key(42)
    k1, k2 = jax.random.split(skey)
    sample_idx = jax.random.randint(k1, (M, NUM_HARD), 0, K_HARD)
    rows = jnp.arange(M)
    r = jax.random.randint(k2, (M, NUM_INBATCH), 0, M - 1)
    inbatch_idx = r + (r >= rows[:, None]).astype(r.dtype)
    idx_pad = jnp.pad(inbatch_idx.astype(jnp.int32),
                      ((0, 0), (0, IDX_PAD - NUM_INBATCH)))

    qn, qb, kb, l0 = _normalize(q, k)
    hard_rows = _hard_gather(item_table, ids_flat)
    s = _similarity(qb, kb)
    in_part = _inbatch_sums(s, idx_pad.reshape(M * IDX_PAD))
    num, den = _finalize(
        qn,
        hard_rows.reshape(M, K_HARD, D),
        sample_idx.astype(jnp.int32),
        l0,
        in_part.reshape(M, 16),
        loss_mask.reshape(M),
    )
    return num[0, 0] / den[0, 0]


# speedup vs baseline: 1.8731x; 1.2529x over previous
import functools

import jax
import jax.numpy as jnp
from jax import lax
from jax.experimental import pallas as pl
from jax.experimental.pallas import tpu as pltpu
from jax.experimental.pallas import tpu_sc as plsc

B, L, D = 1024, 20, 128
M = B * L
K_HARD = 16
T_TOTAL = 512
NUM_HARD = 153
NUM_INBATCH = T_TOTAL - NUM_HARD
INV_TEMP = 20.0
IDX_PAD = 368
N_GROUPS = IDX_PAD // 16
TAIL_VALID = NUM_INBATCH - (N_GROUPS - 1) * 16

_SC_INFO = plsc.get_sparse_core_info()
NC = _SC_INFO.num_cores
NS = _SC_INFO.num_subcores
NW = NC * NS


def _norm_body(q_ref, k_ref, qn_ref, qb_ref, kb_ref, l0_ref):
    q = q_ref[...]
    k = k_ref[...]
    qn = q / jnp.maximum(jnp.sqrt(jnp.sum(q * q, axis=1, keepdims=True)), 1e-12)
    kn = k / jnp.maximum(jnp.sqrt(jnp.sum(k * k, axis=1, keepdims=True)), 1e-12)
    qn_ref[...] = qn
    qb_ref[...] = qn.astype(jnp.bfloat16)
    kb_ref[...] = kn.astype(jnp.bfloat16)
    l0_ref[...] = jnp.sum(qn * kn, axis=1) * INV_TEMP


def _normalize(q, k):
    bm = 2048
    return pl.pallas_call(
        _norm_body,
        grid=(M // bm,),
        in_specs=[
            pl.BlockSpec((bm, D), lambda i: (i, 0)),
            pl.BlockSpec((bm, D), lambda i: (i, 0)),
        ],
        out_specs=[
            pl.BlockSpec((bm, D), lambda i: (i, 0)),
            pl.BlockSpec((bm, D), lambda i: (i, 0)),
            pl.BlockSpec((bm, D), lambda i: (i, 0)),
            pl.BlockSpec((bm,), lambda i: (i,)),
        ],
        out_shape=[
            jax.ShapeDtypeStruct((M, D), jnp.float32),
            jax.ShapeDtypeStruct((M, D), jnp.bfloat16),
            jax.ShapeDtypeStruct((M, D), jnp.bfloat16),
            jax.ShapeDtypeStruct((M,), jnp.float32),
        ],
    )(q, k)


_HALF = M // 2


def _rne_bf16_bits(x):
    u = pltpu.bitcast(x, jnp.uint32)
    return u + jnp.uint32(0x7FFF) + ((u >> jnp.uint32(16)) & jnp.uint32(1))


def _matmul_body(a_ref, bl_ref, bh_ref, o_ref):
    a = a_ref[...]
    dn = (((1,), (1,)), ((), ()))
    lo = lax.dot_general(a, bl_ref[...], dn,
                         preferred_element_type=jnp.float32) * INV_TEMP
    hi = lax.dot_general(a, bh_ref[...], dn,
                         preferred_element_type=jnp.float32) * INV_TEMP
    rl = _rne_bf16_bits(lo)
    rh = _rne_bf16_bits(hi)
    word = (rl >> jnp.uint32(16)) | (rh & jnp.uint32(0xFFFF0000))
    o_ref[...] = pltpu.bitcast(word, jnp.int32)


def _similarity(qb, kb):
    bm, bn = 1024, 1024
    return pl.pallas_call(
        _matmul_body,
        grid=(M // bm, _HALF // bn),
        in_specs=[
            pl.BlockSpec((bm, D), lambda i, j: (i, 0)),
            pl.BlockSpec((bn, D), lambda i, j: (j, 0)),
            pl.BlockSpec((bn, D), lambda i, j: (j + _HALF // bn, 0)),
        ],
        out_specs=pl.BlockSpec((bm, bn), lambda i, j: (i, j)),
        out_shape=jax.ShapeDtypeStruct((M, _HALF), jnp.int32),
    )(qb, kb, kb)


_G_TOTAL = M * K_HARD
_G_PER_W = _G_TOTAL // NW
_G_CHUNK = 128
_G_STEPS = _G_PER_W // _G_CHUNK


def _hard_gather_body(table_hbm, ids_hbm, out_hbm, idx_v, rows_v, sem):
    wid = lax.axis_index("s") * NC + lax.axis_index("c")
    wbase = wid * _G_PER_W

    def step(c, _):
        base = wbase + c * _G_CHUNK
        pltpu.sync_copy(ids_hbm.at[pl.ds(base, _G_CHUNK)], idx_v)
        pltpu.async_copy(table_hbm.at[idx_v], rows_v, sem).wait()
        pltpu.sync_copy(rows_v, out_hbm.at[pl.ds(base, _G_CHUNK)])
        return _

    lax.fori_loop(0, _G_STEPS, step, None)


def _hard_gather(item_table, ids_flat):
    mesh = plsc.VectorSubcoreMesh(core_axis_name="c", subcore_axis_name="s")
    f = pl.kernel(
        _hard_gather_body,
        out_type=jax.ShapeDtypeStruct((_G_TOTAL, D), jnp.float32),
        mesh=mesh,
        scratch_types=[
            pltpu.VMEM((_G_CHUNK,), jnp.int32),
            pltpu.VMEM((_G_CHUNK, D), jnp.float32),
            pltpu.SemaphoreType.DMA,
        ],
    )
    return f(item_table, ids_flat)


_R_PER_W = M // NW
_R_GRP = 64
_R_NGRP = _R_PER_W // _R_GRP


def _inbatch_body(s_hbm, idx_hbm, out_hbm, srow_v, idx_v, acc_v,
                  sem0, sem1):
    wid = lax.axis_index("s") * NC + lax.axis_index("c")
    wbase = wid * _R_PER_W
    iota16 = lax.iota(jnp.int32, 16)
    tail_mask = iota16 < TAIL_VALID
    sems = (sem0, sem1)
    bufs = (srow_v.at[pl.ds(0, _HALF)], srow_v.at[pl.ds(_HALF, _HALF)])

    pltpu.async_copy(s_hbm.at[wbase], bufs[0], sem0)

    def group(g, _):
        grp_base = g * _R_GRP
        pltpu.sync_copy(
            idx_hbm.at[pl.ds((wbase + grp_base) * IDX_PAD, _R_GRP * IDX_PAD)],
            idx_v)

        def pair(ph, _):
            for sub in range(2):
                gi = grp_base + ph * 2 + sub
                m = wbase + gi

                @pl.when(gi + 1 < _R_PER_W)
                def _prefetch():
                    pltpu.async_copy(s_hbm.at[m + 1], bufs[1 - sub],
                                     sems[1 - sub])

                pltpu.make_async_copy(s_hbm.at[0], bufs[sub],
                                      sems[sub]).wait()
                li = (ph * 2 + sub) * IDX_PAD
                acc = jnp.zeros((16,), jnp.float32)
                off = sub * _HALF
                for g16 in range(N_GROUPS):
                    iv = idx_v[pl.ds(li + g16 * 16, 16)]
                    hi_m = iv >= _HALF
                    wv = jnp.where(hi_m, iv - _HALF, iv) + off
                    w = plsc.load_gather(srow_v, [wv])
                    vb = jnp.where(hi_m, w & jnp.int32(-65536),
                                   w << jnp.int32(16))
                    e = jnp.exp(plsc.bitcast(vb, jnp.float32))
                    if g16 == N_GROUPS - 1:
                        e = jnp.where(tail_mask, e, 0.0)
                    acc = acc + e
                acc_v[pl.ds((ph * 2 + sub) * 16, 16)] = acc
            return _

        lax.fori_loop(0, _R_GRP // 2, pair, None)
        pltpu.sync_copy(acc_v,
                        out_hbm.at[pl.ds((wbase + grp_base) * 16, _R_GRP * 16)])
        return _

    lax.fori_loop(0, _R_NGRP, group, None)


def _inbatch_sums(s, idx_pad_flat):
    mesh = plsc.VectorSubcoreMesh(core_axis_name="c", subcore_axis_name="s")
    f = pl.kernel(
        _inbatch_body,
        out_type=jax.ShapeDtypeStruct((M * 16,), jnp.float32),
        mesh=mesh,
        scratch_types=[
            pltpu.VMEM((2 * _HALF,), jnp.int32),
            pltpu.VMEM((_R_GRP * IDX_PAD,), jnp.int32),
            pltpu.VMEM((_R_GRP * 16,), jnp.float32),
            pltpu.SemaphoreType.DMA,
            pltpu.SemaphoreType.DMA,
        ],
        compiler_params=pltpu.CompilerParams(needs_layout_passes=False),
    )
    return f(s, idx_pad_flat)


def _final_body(qn_ref, h_ref, sidx_ref, l0_ref, inp_ref, mask_ref,
                num_ref, den_ref):
    @pl.when(pl.program_id(0) == 0)
    def _init():
        num_ref[...] = jnp.zeros((1, 1), jnp.float32)
        den_ref[...] = jnp.zeros((1, 1), jnp.float32)

    qn = qn_ref[...]
    h = h_ref[...]
    hn = h / jnp.maximum(
        jnp.sqrt(jnp.sum(h * h, axis=2, keepdims=True)), 1e-12)
    l16 = jnp.sum(qn[:, None, :] * hn, axis=2) * INV_TEMP
    e16 = jnp.exp(l16)
    sidx = sidx_ref[...]
    hard_sum = jnp.zeros(e16.shape[:1], jnp.float32)
    for j in range(K_HARD):
        cnt = jnp.sum((sidx == j).astype(jnp.float32), axis=1)
        hard_sum = hard_sum + cnt * e16[:, j]
    in_sum = jnp.sum(inp_ref[...], axis=1)
    l0 = l0_ref[...]
    mask = (mask_ref[...] > 0.5).astype(jnp.float32)
    loss = jnp.log(jnp.exp(l0) + hard_sum + in_sum) - l0
    num_ref[...] += jnp.sum(loss * mask).reshape(1, 1)
    den_ref[...] += jnp.sum(mask).reshape(1, 1)


def _finalize(qn, hard_rows, sample_idx, l0, in_part, mask_flat):
    bf = 1024
    return pl.pallas_call(
        _final_body,
        grid=(M // bf,),
        in_specs=[
            pl.BlockSpec((bf, D), lambda i: (i, 0)),
            pl.BlockSpec((bf, K_HARD, D), lambda i: (i, 0, 0)),
            pl.BlockSpec((bf, NUM_HARD), lambda i: (i, 0)),
            pl.BlockSpec((bf,), lambda i: (i,)),
            pl.BlockSpec((bf, 16), lambda i: (i, 0)),
            pl.BlockSpec((bf,), lambda i: (i,)),
        ],
        out_specs=[
            pl.BlockSpec((1, 1), lambda i: (0, 0)),
            pl.BlockSpec((1, 1), lambda i: (0, 0)),
        ],
        out_shape=[
            jax.ShapeDtypeStruct((1, 1), jnp.float32),
            jax.ShapeDtypeStruct((1, 1), jnp.float32),
        ],
    )(qn, hard_rows, sample_idx, l0, in_part, mask_flat)


def kernel(user_embs, pos_embs, neg_embs, hard_negatives, loss_mask, item_table):
    del neg_embs
    q = user_embs.reshape(M, D)
    k = pos_embs.reshape(M, D)
    ids_flat = hard_negatives.reshape(M * K_HARD)

    skey = jax.random.key(42)
    k1, k2 = jax.random.split(skey)
    sample_idx = jax.random.randint(k1, (M, NUM_HARD), 0, K_HARD)
    rows = jnp.arange(M)
    r = jax.random.randint(k2, (M, NUM_INBATCH), 0, M - 1)
    inbatch_idx = r + (r >= rows[:, None]).astype(r.dtype)
    idx_pad = jnp.pad(inbatch_idx.astype(jnp.int32),
                      ((0, 0), (0, IDX_PAD - NUM_INBATCH)))

    qn, qb, kb, l0 = _normalize(q, k)
    hard_rows = _hard_gather(item_table, ids_flat)
    s = _similarity(qb, kb)
    in_part = _inbatch_sums(s, idx_pad.reshape(M * IDX_PAD))
    num, den = _finalize(
        qn,
        hard_rows.reshape(M, K_HARD, D),
        sample_idx.astype(jnp.int32),
        l0,
        in_part.reshape(M, 16),
        loss_mask.reshape(M),
    )
    return num[0, 0] / den[0, 0]


# speedup vs baseline: 2.1143x; 1.1288x over previous
import functools

import jax
import jax.numpy as jnp
from jax import lax
from jax.experimental import pallas as pl
from jax.experimental.pallas import tpu as pltpu
from jax.experimental.pallas import tpu_sc as plsc

B, L, D = 1024, 20, 128
M = B * L
K_HARD = 16
T_TOTAL = 512
NUM_HARD = 153
NUM_INBATCH = T_TOTAL - NUM_HARD
INV_TEMP = 20.0
IDX_PAD = 368
N_GROUPS = IDX_PAD // 16
TAIL_VALID = NUM_INBATCH - (N_GROUPS - 1) * 16

_SC_INFO = plsc.get_sparse_core_info()
NC = _SC_INFO.num_cores
NS = _SC_INFO.num_subcores
NW = NC * NS


def _norm_body(q_ref, k_ref, qn_ref, qb_ref, kb_ref, l0_ref):
    q = q_ref[...]
    k = k_ref[...]
    qn = q / jnp.maximum(jnp.sqrt(jnp.sum(q * q, axis=1, keepdims=True)), 1e-12)
    kn = k / jnp.maximum(jnp.sqrt(jnp.sum(k * k, axis=1, keepdims=True)), 1e-12)
    qn_ref[...] = qn
    qb_ref[...] = qn.astype(jnp.bfloat16)
    kb_ref[...] = kn.astype(jnp.bfloat16)
    l0_ref[...] = jnp.sum(qn * kn, axis=1) * INV_TEMP


def _normalize(q, k):
    bm = 2048
    return pl.pallas_call(
        _norm_body,
        grid=(M // bm,),
        in_specs=[
            pl.BlockSpec((bm, D), lambda i: (i, 0)),
            pl.BlockSpec((bm, D), lambda i: (i, 0)),
        ],
        out_specs=[
            pl.BlockSpec((bm, D), lambda i: (i, 0)),
            pl.BlockSpec((bm, D), lambda i: (i, 0)),
            pl.BlockSpec((bm, D), lambda i: (i, 0)),
            pl.BlockSpec((bm,), lambda i: (i,)),
        ],
        out_shape=[
            jax.ShapeDtypeStruct((M, D), jnp.float32),
            jax.ShapeDtypeStruct((M, D), jnp.bfloat16),
            jax.ShapeDtypeStruct((M, D), jnp.bfloat16),
            jax.ShapeDtypeStruct((M,), jnp.float32),
        ],
    )(q, k)


_HALF = M // 2


def _rne_bf16_bits(x):
    u = pltpu.bitcast(x, jnp.uint32)
    return u + jnp.uint32(0x7FFF) + ((u >> jnp.uint32(16)) & jnp.uint32(1))


def _matmul_body(a_ref, bl_ref, bh_ref, o_ref):
    a = a_ref[...]
    dn = (((1,), (1,)), ((), ()))
    lo = lax.dot_general(a, bl_ref[...], dn,
                         preferred_element_type=jnp.float32) * INV_TEMP
    hi = lax.dot_general(a, bh_ref[...], dn,
                         preferred_element_type=jnp.float32) * INV_TEMP
    rl = _rne_bf16_bits(lo)
    rh = _rne_bf16_bits(hi)
    word = (rl >> jnp.uint32(16)) | (rh & jnp.uint32(0xFFFF0000))
    o_ref[...] = pltpu.bitcast(word, jnp.int32)


def _similarity(qb, kb):
    bm, bn = 1024, 1024
    return pl.pallas_call(
        _matmul_body,
        grid=(M // bm, _HALF // bn),
        in_specs=[
            pl.BlockSpec((bm, D), lambda i, j: (i, 0)),
            pl.BlockSpec((bn, D), lambda i, j: (j, 0)),
            pl.BlockSpec((bn, D), lambda i, j: (j + _HALF // bn, 0)),
        ],
        out_specs=pl.BlockSpec((bm, bn), lambda i, j: (i, j)),
        out_shape=jax.ShapeDtypeStruct((M, _HALF), jnp.int32),
    )(qb, kb, kb)


_G_TOTAL = M * K_HARD
_G_PER_W = _G_TOTAL // NW
_G_CHUNK = 320
_G_STEPS = _G_PER_W // _G_CHUNK


def _hard_gather_body(table_hbm, ids_hbm, out_hbm, idx_v, rows_v,
                      semg0, semg1, semw0, semw1):
    wid = lax.axis_index("s") * NC + lax.axis_index("c")
    wbase = wid * _G_PER_W
    semg = (semg0, semg1)
    semw = (semw0, semw1)
    bufi = (idx_v.at[pl.ds(0, _G_CHUNK)], idx_v.at[pl.ds(_G_CHUNK, _G_CHUNK)])
    bufr = (rows_v.at[pl.ds(0, _G_CHUNK)], rows_v.at[pl.ds(_G_CHUNK, _G_CHUNK)])

    pltpu.sync_copy(ids_hbm.at[pl.ds(wbase, _G_CHUNK)], bufi[0])
    pltpu.async_copy(table_hbm.at[bufi[0]], bufr[0], semg0)

    def pair(ph, _):
        for sub in range(2):
            c = ph * 2 + sub

            @pl.when(c + 1 < _G_STEPS)
            def _next():
                nb = wbase + (c + 1) * _G_CHUNK
                pltpu.sync_copy(ids_hbm.at[pl.ds(nb, _G_CHUNK)], bufi[1 - sub])

                @pl.when(c >= 1)
                def _drain_w():
                    pltpu.make_async_copy(
                        bufr[1 - sub],
                        out_hbm.at[pl.ds(wbase, _G_CHUNK)],
                        semw[1 - sub]).wait()

                pltpu.async_copy(table_hbm.at[bufi[1 - sub]], bufr[1 - sub],
                                 semg[1 - sub])

            pltpu.make_async_copy(table_hbm.at[bufi[sub]], bufr[sub],
                                  semg[sub]).wait()
            pltpu.async_copy(bufr[sub],
                             out_hbm.at[pl.ds(wbase + c * _G_CHUNK, _G_CHUNK)],
                             semw[sub])
        return _

    lax.fori_loop(0, _G_STEPS // 2, pair, None)
    for p in range(2):
        pltpu.make_async_copy(bufr[p], out_hbm.at[pl.ds(wbase, _G_CHUNK)],
                              semw[p]).wait()


def _hard_gather(item_table, ids_flat):
    mesh = plsc.VectorSubcoreMesh(core_axis_name="c", subcore_axis_name="s")
    f = pl.kernel(
        _hard_gather_body,
        out_type=jax.ShapeDtypeStruct((_G_TOTAL, D), jnp.float32),
        mesh=mesh,
        scratch_types=[
            pltpu.VMEM((2 * _G_CHUNK,), jnp.int32),
            pltpu.VMEM((2 * _G_CHUNK, D), jnp.float32),
            pltpu.SemaphoreType.DMA,
            pltpu.SemaphoreType.DMA,
            pltpu.SemaphoreType.DMA,
            pltpu.SemaphoreType.DMA,
        ],
    )
    return f(item_table, ids_flat)


_R_PER_W = M // NW
_R_GRP = 64
_R_NGRP = _R_PER_W // _R_GRP


_R_NBUF = 4


def _inbatch_body(s_hbm, idx_hbm, out_hbm, srow_v, idx_v, acc_v,
                  sem0, sem1, sem2, sem3):
    wid = lax.axis_index("s") * NC + lax.axis_index("c")
    wbase = wid * _R_PER_W
    iota16 = lax.iota(jnp.int32, 16)
    tail_mask = iota16 < TAIL_VALID
    sems = (sem0, sem1, sem2, sem3)
    bufs = tuple(srow_v.at[pl.ds(p * _HALF, _HALF)] for p in range(_R_NBUF))

    for p in range(_R_NBUF - 1):
        pltpu.async_copy(s_hbm.at[wbase + p], bufs[p], sems[p])

    def group(g, _):
        grp_base = g * _R_GRP
        pltpu.sync_copy(
            idx_hbm.at[pl.ds((wbase + grp_base) * IDX_PAD, _R_GRP * IDX_PAD)],
            idx_v)

        def quad(ph, _):
            for sub in range(_R_NBUF):
                gi = grp_base + ph * _R_NBUF + sub
                m = wbase + gi

                @pl.when(gi + (_R_NBUF - 1) < _R_PER_W)
                def _prefetch():
                    pltpu.async_copy(s_hbm.at[m + (_R_NBUF - 1)],
                                     bufs[(sub + _R_NBUF - 1) % _R_NBUF],
                                     sems[(sub + _R_NBUF - 1) % _R_NBUF])

                pltpu.make_async_copy(s_hbm.at[0], bufs[sub],
                                      sems[sub]).wait()
                li = (ph * _R_NBUF + sub) * IDX_PAD
                acc = jnp.zeros((16,), jnp.float32)
                off = sub * _HALF
                for g16 in range(N_GROUPS):
                    iv = idx_v[pl.ds(li + g16 * 16, 16)]
                    hi_m = iv >= _HALF
                    wv = jnp.where(hi_m, iv - _HALF, iv) + off
                    w = plsc.load_gather(srow_v, [wv])
                    vb = jnp.where(hi_m, w & jnp.int32(-65536),
                                   w << jnp.int32(16))
                    e = jnp.exp(plsc.bitcast(vb, jnp.float32))
                    if g16 == N_GROUPS - 1:
                        e = jnp.where(tail_mask, e, 0.0)
                    acc = acc + e
                acc_v[pl.ds((ph * _R_NBUF + sub) * 16, 16)] = acc
            return _

        lax.fori_loop(0, _R_GRP // _R_NBUF, quad, None)
        pltpu.sync_copy(acc_v,
                        out_hbm.at[pl.ds((wbase + grp_base) * 16, _R_GRP * 16)])
        return _

    lax.fori_loop(0, _R_NGRP, group, None)


def _inbatch_sums(s, idx_pad_flat):
    mesh = plsc.VectorSubcoreMesh(core_axis_name="c", subcore_axis_name="s")
    f = pl.kernel(
        _inbatch_body,
        out_type=jax.ShapeDtypeStruct((M * 16,), jnp.float32),
        mesh=mesh,
        scratch_types=[
            pltpu.VMEM((_R_NBUF * _HALF,), jnp.int32),
            pltpu.VMEM((_R_GRP * IDX_PAD,), jnp.int32),
            pltpu.VMEM((_R_GRP * 16,), jnp.float32),
            pltpu.SemaphoreType.DMA,
            pltpu.SemaphoreType.DMA,
            pltpu.SemaphoreType.DMA,
            pltpu.SemaphoreType.DMA,
        ],
        compiler_params=pltpu.CompilerParams(needs_layout_passes=False),
    )
    return f(s, idx_pad_flat)


def _final_body(qn_ref, h_ref, sidx_ref, l0_ref, inp_ref, mask_ref,
                num_ref, den_ref):
    @pl.when(pl.program_id(0) == 0)
    def _init():
        num_ref[...] = jnp.zeros((1, 1), jnp.float32)
        den_ref[...] = jnp.zeros((1, 1), jnp.float32)

    qn = qn_ref[...]
    h = h_ref[...]
    hn = h / jnp.maximum(
        jnp.sqrt(jnp.sum(h * h, axis=2, keepdims=True)), 1e-12)
    l16 = jnp.sum(qn[:, None, :] * hn, axis=2) * INV_TEMP
    e16 = jnp.exp(l16)
    sidx = sidx_ref[...]
    hard_sum = jnp.zeros(e16.shape[:1], jnp.float32)
    for j in range(K_HARD):
        cnt = jnp.sum((sidx == j).astype(jnp.float32), axis=1)
        hard_sum = hard_sum + cnt * e16[:, j]
    in_sum = jnp.sum(inp_ref[...], axis=1)
    l0 = l0_ref[...]
    mask = (mask_ref[...] > 0.5).astype(jnp.float32)
    loss = jnp.log(jnp.exp(l0) + hard_sum + in_sum) - l0
    num_ref[...] += jnp.sum(loss * mask).reshape(1, 1)
    den_ref[...] += jnp.sum(mask).reshape(1, 1)


def _finalize(qn, hard_rows, sample_idx, l0, in_part, mask_flat):
    bf = 1024
    return pl.pallas_call(
        _final_body,
        grid=(M // bf,),
        in_specs=[
            pl.BlockSpec((bf, D), lambda i: (i, 0)),
            pl.BlockSpec((bf, K_HARD, D), lambda i: (i, 0, 0)),
            pl.BlockSpec((bf, NUM_HARD), lambda i: (i, 0)),
            pl.BlockSpec((bf,), lambda i: (i,)),
            pl.BlockSpec((bf, 16), lambda i: (i, 0)),
            pl.BlockSpec((bf,), lambda i: (i,)),
        ],
        out_specs=[
            pl.BlockSpec((1, 1), lambda i: (0, 0)),
            pl.BlockSpec((1, 1), lambda i: (0, 0)),
        ],
        out_shape=[
            jax.ShapeDtypeStruct((1, 1), jnp.float32),
            jax.ShapeDtypeStruct((1, 1), jnp.float32),
        ],
    )(qn, hard_rows, sample_idx, l0, in_part, mask_flat)


def kernel(user_embs, pos_embs, neg_embs, hard_negatives, loss_mask, item_table):
    del neg_embs
    q = user_embs.reshape(M, D)
    k = pos_embs.reshape(M, D)
    ids_flat = hard_negatives.reshape(M * K_HARD)

    skey = jax.random.key(42)
    k1, k2 = jax.random.split(skey)
    sample_idx = jax.random.randint(k1, (M, NUM_HARD), 0, K_HARD)
    rows = jnp.arange(M)
    r = jax.random.randint(k2, (M, NUM_INBATCH), 0, M - 1)
    inbatch_idx = r + (r >= rows[:, None]).astype(r.dtype)
    idx_pad = jnp.pad(inbatch_idx.astype(jnp.int32),
                      ((0, 0), (0, IDX_PAD - NUM_INBATCH)))

    qn, qb, kb, l0 = _normalize(q, k)
    hard_rows = _hard_gather(item_table, ids_flat)
    s = _similarity(qb, kb)
    in_part = _inbatch_sums(s, idx_pad.reshape(M * IDX_PAD))
    num, den = _finalize(
        qn,
        hard_rows.reshape(M, K_HARD, D),
        sample_idx.astype(jnp.int32),
        l0,
        in_part.reshape(M, 16),
        loss_mask.reshape(M),
    )
    return num[0, 0] / den[0, 0]
